# baseline (speedup 1.0000x reference)
import jax
import jax.numpy as jnp
from jax.experimental import pallas as pl
from jax.experimental.pallas import tpu as pltpu
from jax.experimental.pallas import tpu_sc as plsc

VOCAB = 50
POS_DIM = 64
WINDOW = 128


def _sc_gather(rep_table, idx_flat, total):
    mesh = plsc.VectorSubcoreMesh(core_axis_name="core", subcore_axis_name="subcore")

    @pl.kernel(
        out_type=jax.ShapeDtypeStruct((total, 2 * POS_DIM), rep_table.dtype),
        mesh=mesh,
    )
    def gather_kernel(table_hbm, idx_hbm, out_hbm):
        def body(idx_vmem, out_vmem):
            pltpu.sync_copy(table_hbm.at[idx_vmem.at[0]], out_vmem)

        pltpu.emit_pipeline(
            body,
            grid=(total // WINDOW,),
            in_specs=[pl.BlockSpec((1, WINDOW), index_map=lambda i: (0, i))],
            out_specs=[
                pl.BlockSpec((WINDOW, 2 * POS_DIM), index_map=lambda i: (i, 0))
            ],
            core_axis_name=("core", "subcore"),
            dimension_semantics=(pltpu.PARALLEL,),
        )(idx_hbm, out_hbm)

    return gather_kernel(rep_table, idx_flat)


def kernel(indices, table):
    batch, seq_len = indices.shape
    total = batch * seq_len
    rep_table = jnp.concatenate([table, table], axis=1)
    idx_flat = indices.astype(jnp.int32).reshape(1, total)
    wide = _sc_gather(rep_table, idx_flat, total)
    return wide.reshape(batch, seq_len, 2 * POS_DIM)[:, :, :POS_DIM]


# speedup vs baseline: 2.8233x; 2.8233x over previous
import jax
import jax.numpy as jnp
from jax.experimental import pallas as pl
from jax.experimental.pallas import tpu as pltpu
from jax.experimental.pallas import tpu_sc as plsc

VOCAB = 50
POS_DIM = 64
WINDOW = 128


def _sc_gather(rep_table, idx_flat, total):
    mesh = plsc.VectorSubcoreMesh(core_axis_name="core", subcore_axis_name="subcore")

    @pl.kernel(
        out_type=jax.ShapeDtypeStruct((total, 2 * POS_DIM), rep_table.dtype),
        mesh=mesh,
    )
    def gather_kernel(table_hbm, idx_hbm, out_hbm):
        def body(idx_vmem, out_vmem):
            pltpu.sync_copy(table_hbm.at[idx_vmem.at[0]], out_vmem)

        pltpu.emit_pipeline(
            body,
            grid=(total // WINDOW,),
            in_specs=[pl.BlockSpec((1, WINDOW), index_map=lambda i: (0, i))],
            out_specs=[
                pl.BlockSpec((WINDOW, 2 * POS_DIM), index_map=lambda i: (i, 0))
            ],
            core_axis_name=("core", "subcore"),
            dimension_semantics=(pltpu.PARALLEL,),
        )(idx_hbm, out_hbm)

    return gather_kernel(rep_table, idx_flat)


SPREAD = 64


def kernel(indices, table):
    batch, seq_len = indices.shape
    total = batch * seq_len
    rep_table = jnp.tile(jnp.concatenate([table, table], axis=1), (SPREAD, 1))
    idx_flat = indices.astype(jnp.int32).reshape(1, total)
    offs = (jax.lax.iota(jnp.int32, total) & (SPREAD - 1)).reshape(1, total)
    idx_flat = idx_flat + VOCAB * offs
    wide = _sc_gather(rep_table, idx_flat, total)
    return wide.reshape(batch, seq_len, 2 * POS_DIM)[:, :, :POS_DIM]


# speedup vs baseline: 3.1411x; 1.1126x over previous
import jax
import jax.numpy as jnp
from jax.experimental import pallas as pl
from jax.experimental.pallas import tpu as pltpu
from jax.experimental.pallas import tpu_sc as plsc

VOCAB = 50
POS_DIM = 64
WINDOW = 128


def _sc_gather(slab_table, idx_flat, n_pairs):
    mesh = plsc.VectorSubcoreMesh(core_axis_name="core", subcore_axis_name="subcore")

    @pl.kernel(
        out_type=jax.ShapeDtypeStruct((n_pairs, 2, 2 * POS_DIM), slab_table.dtype),
        mesh=mesh,
    )
    def gather_kernel(table_hbm, idx_hbm, out_hbm):
        def body(idx_vmem, out_vmem):
            pltpu.sync_copy(table_hbm.at[idx_vmem.at[0]], out_vmem)

        pltpu.emit_pipeline(
            body,
            grid=(n_pairs // WINDOW,),
            in_specs=[pl.BlockSpec((1, WINDOW), index_map=lambda i: (0, i))],
            out_specs=[
                pl.BlockSpec(
                    (WINDOW, 2, 2 * POS_DIM), index_map=lambda i: (i, 0, 0)
                )
            ],
            core_axis_name=("core", "subcore"),
            dimension_semantics=(pltpu.PARALLEL,),
        )(idx_hbm, out_hbm)

    return gather_kernel(slab_table, idx_flat)


def kernel(indices, table):
    batch, seq_len = indices.shape
    n_pairs = batch * seq_len // 2

    rep = jnp.concatenate([table, table], axis=1)
    slab_table = jnp.stack(
        [
            jnp.broadcast_to(rep[:, None, :], (VOCAB, VOCAB, 2 * POS_DIM)),
            jnp.broadcast_to(rep[None, :, :], (VOCAB, VOCAB, 2 * POS_DIM)),
        ],
        axis=2,
    ).reshape(VOCAB * VOCAB, 2, 2 * POS_DIM)

    idx = indices.astype(jnp.int32)
    pair_idx = (idx[:, 0::2] * VOCAB + idx[:, 1::2]).reshape(1, n_pairs)

    wide = _sc_gather(slab_table, pair_idx, n_pairs)
    return wide.reshape(batch, seq_len, 2 * POS_DIM)[:, :, :POS_DIM]


# speedup vs baseline: 3.2379x; 1.0308x over previous
import jax
import jax.numpy as jnp
from jax.experimental import pallas as pl
from jax.experimental.pallas import tpu as pltpu
from jax.experimental.pallas import tpu_sc as plsc

VOCAB = 50
POS_DIM = 64
WINDOW = 128


def _sc_gather(slab_table, idx_flat, n_pairs):
    mesh = plsc.VectorSubcoreMesh(core_axis_name="core", subcore_axis_name="subcore")

    @pl.kernel(
        out_type=jax.ShapeDtypeStruct((n_pairs, 2, 2 * POS_DIM), slab_table.dtype),
        mesh=mesh,
    )
    def gather_kernel(table_hbm, idx_hbm, out_hbm):
        def body(idx_vmem, out_vmem):
            pltpu.sync_copy(table_hbm.at[idx_vmem.at[0]], out_vmem)

        pltpu.emit_pipeline(
            body,
            grid=(n_pairs // WINDOW,),
            in_specs=[pl.BlockSpec((1, WINDOW), index_map=lambda i: (0, i))],
            out_specs=[
                pl.BlockSpec(
                    (WINDOW, 2, 2 * POS_DIM), index_map=lambda i: (i, 0, 0)
                )
            ],
            core_axis_name=("core", "subcore"),
            dimension_semantics=(pltpu.PARALLEL,),
        )(idx_hbm, out_hbm)

    return gather_kernel(slab_table, idx_flat)


def kernel(indices, table):
    batch, seq_len = indices.shape
    n_pairs = batch * seq_len // 2

    rep = jnp.concatenate([table, table], axis=1)
    slab_table = jnp.stack(
        [
            jnp.broadcast_to(rep[:, None, :], (VOCAB, VOCAB, 2 * POS_DIM)),
            jnp.broadcast_to(rep[None, :, :], (VOCAB, VOCAB, 2 * POS_DIM)),
        ],
        axis=2,
    ).reshape(VOCAB * VOCAB, 2, 2 * POS_DIM)

    slab_table = jnp.tile(slab_table, (4, 1, 1))
    idx = indices.astype(jnp.int32)
    pair_idx = (idx[:, 0::2] * VOCAB + idx[:, 1::2]).reshape(1, n_pairs)
    offs = (jax.lax.iota(jnp.int32, n_pairs) & 3).reshape(1, n_pairs)
    pair_idx = pair_idx + (VOCAB * VOCAB) * offs

    wide = _sc_gather(slab_table, pair_idx, n_pairs)
    return wide.reshape(batch, seq_len, 2 * POS_DIM)[:, :, :POS_DIM]


# speedup vs baseline: 3.2765x; 1.0119x over previous
import jax
import jax.numpy as jnp
from jax.experimental import pallas as pl
from jax.experimental.pallas import tpu as pltpu
from jax.experimental.pallas import tpu_sc as plsc

VOCAB = 50
POS_DIM = 64
WINDOW = 128


def _sc_gather(slab_table, idx_flat, n_pairs):
    mesh = plsc.VectorSubcoreMesh(core_axis_name="core", subcore_axis_name="subcore")

    @pl.kernel(
        out_type=jax.ShapeDtypeStruct((n_pairs, 2, 2 * POS_DIM), slab_table.dtype),
        mesh=mesh,
        scratch_types=[pltpu.SemaphoreType.DMA, pltpu.SemaphoreType.DMA],
    )
    def gather_kernel(table_hbm, idx_hbm, out_hbm, sem_a, sem_b):
        half = WINDOW // 2

        def body(idx_vmem, out_vmem):
            h1 = pltpu.async_copy(
                table_hbm.at[idx_vmem.at[0, pl.ds(0, half)]],
                out_vmem.at[pl.ds(0, half)],
                sem_a,
            )
            h2 = pltpu.async_copy(
                table_hbm.at[idx_vmem.at[0, pl.ds(half, half)]],
                out_vmem.at[pl.ds(half, half)],
                sem_b,
            )
            h1.wait()
            h2.wait()

        pltpu.emit_pipeline(
            body,
            grid=(n_pairs // WINDOW,),
            in_specs=[pl.BlockSpec((1, WINDOW), index_map=lambda i: (0, i))],
            out_specs=[
                pl.BlockSpec(
                    (WINDOW, 2, 2 * POS_DIM), index_map=lambda i: (i, 0, 0)
                )
            ],
            core_axis_name=("core", "subcore"),
            dimension_semantics=(pltpu.PARALLEL,),
        )(idx_hbm, out_hbm)

    return gather_kernel(slab_table, idx_flat)


def kernel(indices, table):
    batch, seq_len = indices.shape
    n_pairs = batch * seq_len // 2

    rep = jnp.concatenate([table, table], axis=1)
    slab_table = jnp.stack(
        [
            jnp.broadcast_to(rep[:, None, :], (VOCAB, VOCAB, 2 * POS_DIM)),
            jnp.broadcast_to(rep[None, :, :], (VOCAB, VOCAB, 2 * POS_DIM)),
        ],
        axis=2,
    ).reshape(VOCAB * VOCAB, 2, 2 * POS_DIM)

    slab_table = jnp.tile(slab_table, (4, 1, 1))
    idx = indices.astype(jnp.int32)
    pair_idx = (idx[:, 0::2] * VOCAB + idx[:, 1::2]).reshape(1, n_pairs)
    offs = (jax.lax.iota(jnp.int32, n_pairs) & 3).reshape(1, n_pairs)
    pair_idx = pair_idx + (VOCAB * VOCAB) * offs

    wide = _sc_gather(slab_table, pair_idx, n_pairs)
    return wide.reshape(batch, seq_len, 2 * POS_DIM)[:, :, :POS_DIM]
